# Initial kernel scaffold; baseline (speedup 1.0000x reference)
import jax
import jax.numpy as jnp
from jax.experimental import pallas as pl


def kernel(p, f, index, offset, params):
    raise NotImplementedError("write your pallas kernel here")


# speedup vs baseline: 3.5314x; 3.5314x over previous
import functools
import jax
import jax.numpy as jnp
from jax import lax
from jax.experimental import pallas as pl
from jax.experimental.pallas import tpu as pltpu

_EPS = 1e-5
_K = 16
_CB = 4096


def _bn_lanes(x, g, bt):
    mean = jnp.mean(x, axis=1, keepdims=True)
    xc = x - mean
    var = jnp.mean(xc * xc, axis=1, keepdims=True)
    return xc / jnp.sqrt(var + _EPS) * g + bt


def _apply_stats(x, s, g, bt, n):
    mean = s[:, 0:1] / n
    var = s[:, 1:2] / n - mean * mean
    return (x - mean) / jnp.sqrt(var + _EPS) * g + bt


def _dot(a, b):
    return lax.dot_general(a, b, (((1,), (0,)), ((), ())),
                           preferred_element_type=jnp.float32)


def _dot_t(a, b):
    return lax.dot_general(a, b, (((0,), (0,)), ((), ())),
                           preferred_element_type=jnp.float32)


def _stats_of(y):
    return jnp.concatenate(
        [jnp.sum(y, axis=1, keepdims=True),
         jnp.sum(y * y, axis=1, keepdims=True)], axis=1)


def _acc_stats(s_ref, y):
    st = _stats_of(y)

    @pl.when(pl.program_id(0) == 0)
    def _():
        s_ref[...] = st

    @pl.when(pl.program_id(0) != 0)
    def _():
        s_ref[...] = s_ref[...] + st


def _knn_body(qT_ref, pT_ref, idx_ref, *, npts):
    qT = qT_ref[...]
    pT = pT_ref[...]
    sq_q = jnp.sum(qT * qT, axis=0)[:, None]
    sq_p = jnp.sum(pT * pT, axis=0)[None, :]
    d2 = sq_q + sq_p - 2.0 * _dot_t(qT, pT)
    iota = lax.broadcasted_iota(jnp.int32, d2.shape, 1)
    inf = jnp.float32(jnp.inf)
    big = jnp.int32(npts)
    cols = []
    for _ in range(_K):
        m = jnp.min(d2, axis=1, keepdims=True)
        pos = jnp.min(jnp.where(d2 <= m, iota, big), axis=1)
        cols.append(pos)
        d2 = jnp.where(iota == pos[:, None], inf, d2)
    idx_ref[...] = jnp.stack(cols, axis=1)


def _knn(qT, pT, tq=128):
    nq = qT.shape[1]
    npts = pT.shape[1]
    return pl.pallas_call(
        functools.partial(_knn_body, npts=npts),
        grid=(nq // tq,),
        in_specs=[
            pl.BlockSpec((3, tq), lambda i: (0, i)),
            pl.BlockSpec((3, npts), lambda i: (0, 0)),
        ],
        out_specs=pl.BlockSpec((tq, _K), lambda i: (i, 0)),
        out_shape=jax.ShapeDtypeStruct((nq, _K), jnp.int32),
        compiler_params=pltpu.CompilerParams(
            dimension_semantics=("parallel",)),
    )(qT, pT)


def _l1_body(pg_ref, q_ref, W_ref, y_ref, s_ref):
    x = pg_ref[...] - q_ref[...]
    y = _dot(W_ref[...], x)
    y_ref[...] = y
    _acc_stats(s_ref, y)


def _lmid_body(x_ref, s_ref, g_ref, bt_ref, W_ref, y_ref, so_ref, *, n):
    xn = jax.nn.relu(_apply_stats(x_ref[...], s_ref[...],
                                  g_ref[...], bt_ref[...], n))
    y = _dot(W_ref[...], xn)
    y_ref[...] = y
    _acc_stats(so_ref, y)


def _lmax_body(y_ref, add_ref, s_ref, g_ref, bt_ref, o_ref, *, n):
    z = _apply_stats(y_ref[...], s_ref[...], g_ref[...], bt_ref[...], n)
    z = z + add_ref[...]

    @pl.when(pl.program_id(0) == 0)
    def _():
        o_ref[...] = z

    @pl.when(pl.program_id(0) != 0)
    def _():
        o_ref[...] = jnp.maximum(o_ref[...], z)


def _seq_params(grid, cout):
    return dict(
        out_shape=(jax.ShapeDtypeStruct((cout, grid * _CB), jnp.float32),
                   jax.ShapeDtypeStruct((cout, 2), jnp.float32)),
        compiler_params=pltpu.CompilerParams(
            dimension_semantics=("arbitrary",)),
    )


def _mlp_max(pgT, qT, addT, blocks):
    grid = pgT.shape[1] // _CB
    n = float(pgT.shape[1])
    b0, b1, b2 = blocks
    c0, c1, c2 = b0["W"].shape[0], b1["W"].shape[0], b2["W"].shape[0]
    cs = lambda c: pl.BlockSpec((c, _CB), lambda i: (0, i))
    full = lambda r, c: pl.BlockSpec((r, c), lambda i: (0, 0))

    y1, s1 = pl.pallas_call(
        _l1_body, grid=(grid,),
        in_specs=[cs(3), full(3, _CB), full(c0, 3)],
        out_specs=(cs(c0), full(c0, 2)),
        **_seq_params(grid, c0))(pgT, qT, b0["W"])

    y2, s2 = pl.pallas_call(
        functools.partial(_lmid_body, n=n), grid=(grid,),
        in_specs=[cs(c0), full(c0, 2), full(c0, 1), full(c0, 1),
                  full(c1, c0)],
        out_specs=(cs(c1), full(c1, 2)),
        **_seq_params(grid, c1))(
            y1, s1, b0["g"][:, None], b0["bt"][:, None], b1["W"])

    y3, s3 = pl.pallas_call(
        functools.partial(_lmid_body, n=n), grid=(grid,),
        in_specs=[cs(c1), full(c1, 2), full(c1, 1), full(c1, 1),
                  full(c2, c1)],
        out_specs=(cs(c2), full(c2, 2)),
        **_seq_params(grid, c2))(
            y2, s2, b1["g"][:, None], b1["bt"][:, None], b2["W"])

    return pl.pallas_call(
        functools.partial(_lmax_body, n=n), grid=(grid,),
        in_specs=[cs(c2), cs(c2), full(c2, 2), full(c2, 1), full(c2, 1)],
        out_specs=pl.BlockSpec((c2, _CB), lambda i: (0, 0)),
        out_shape=jax.ShapeDtypeStruct((c2, _CB), jnp.float32),
        compiler_params=pltpu.CompilerParams(
            dimension_semantics=("arbitrary",)),
    )(y3, addT, s3, b2["g"][:, None], b2["bt"][:, None])


def _f1_body(fT_ref, W_ref, g_ref, bt_ref, out_ref):
    y = _dot(W_ref[...], fT_ref[...])
    out_ref[...] = jax.nn.relu(_bn_lanes(y, g_ref[...], bt_ref[...]))


def _f1(fT, W, g, bt):
    n = fT.shape[1]
    return pl.pallas_call(
        _f1_body,
        out_shape=jax.ShapeDtypeStruct((128, n), jnp.float32),
    )(fT, W, g, bt)


def _tail1_body(g_ref, gbn_ref, btbn_ref, Wp_ref, gp_ref, btp_ref,
                newf_ref, hpre_ref):
    newf = _bn_lanes(g_ref[...], gbn_ref[...], btbn_ref[...])
    newf_ref[...] = newf
    hpre_ref[...] = jax.nn.relu(
        _bn_lanes(_dot(Wp_ref[...], newf), gp_ref[...], btp_ref[...]))


def _tail2_body(m_ref, id_ref, gbn_ref, btbn_ref,
                Wf0_ref, gf0_ref, btf0_ref, Wf1_ref, gf1_ref, btf1_ref,
                out_ref):
    h = _bn_lanes(m_ref[...], gbn_ref[...], btbn_ref[...])
    h = jax.nn.relu(_bn_lanes(_dot(Wf0_ref[...], h),
                              gf0_ref[...], btf0_ref[...]))
    h = _bn_lanes(_dot(Wf1_ref[...], h), gf1_ref[...], btf1_ref[...])
    out_ref[...] = jax.nn.relu(id_ref[...] + h)


def kernel(p, f, index, offset, params):
    pm = params
    pT = p.T
    new_pT = jnp.take(pT, index, axis=1)
    m = new_pT.shape[1]

    f1T = _f1(f.T, pm["ds_conv1"]["W"],
              pm["ds_conv1"]["g"][:, None], pm["ds_conv1"]["bt"][:, None])

    idx1 = _knn(new_pT, pT)
    idx1_jm = idx1.T.reshape(-1)
    pgT = jnp.take(pT, idx1_jm, axis=1)
    f1gT = jnp.take(f1T, idx1_jm, axis=1)

    g_acc = _mlp_max(pgT, new_pT, f1gT,
                     [pm["ds_conv2_0"], pm["ds_conv2_1"], pm["ds_conv2_2"]])

    newfT, hpreT = pl.pallas_call(
        _tail1_body,
        out_shape=(jax.ShapeDtypeStruct((128, m), jnp.float32),
                   jax.ShapeDtypeStruct((128, m), jnp.float32)),
    )(g_acc, pm["ds_bn"]["g"][:, None], pm["ds_bn"]["bt"][:, None],
      pm["dsa0_pre"]["W"], pm["dsa0_pre"]["g"][:, None],
      pm["dsa0_pre"]["bt"][:, None])

    idx2 = _knn(new_pT, new_pT)
    idx2_jm = idx2.T.reshape(-1)
    pg2T = jnp.take(new_pT, idx2_jm, axis=1)
    hgT = jnp.take(hpreT, idx2_jm, axis=1)

    m_acc = _mlp_max(pg2T, new_pT, hgT,
                     [pm["pe_0"], pm["pe_1"], pm["pe_2"]])

    outT = pl.pallas_call(
        _tail2_body,
        out_shape=jax.ShapeDtypeStruct((128, m), jnp.float32),
    )(m_acc, newfT,
      pm["dsa0_bn"]["g"][:, None], pm["dsa0_bn"]["bt"][:, None],
      pm["dsa0_ffn0"]["W"], pm["dsa0_ffn0"]["g"][:, None],
      pm["dsa0_ffn0"]["bt"][:, None],
      pm["dsa0_ffn1"]["W"], pm["dsa0_ffn1"]["g"][:, None],
      pm["dsa0_ffn1"]["bt"][:, None])

    return new_pT.T, outT.T


# speedup vs baseline: 3.6927x; 1.0457x over previous
import functools
import jax
import jax.numpy as jnp
from jax import lax
from jax.experimental import pallas as pl
from jax.experimental.pallas import tpu as pltpu

_EPS = 1e-5
_K = 16
_CB = 4096


def _bn_lanes(x, g, bt):
    mean = jnp.mean(x, axis=1, keepdims=True)
    xc = x - mean
    var = jnp.mean(xc * xc, axis=1, keepdims=True)
    return xc / jnp.sqrt(var + _EPS) * g + bt


def _apply_stats(x, s, g, bt, n):
    mean = s[:, 0:1] / n
    var = s[:, 1:2] / n - mean * mean
    return (x - mean) / jnp.sqrt(var + _EPS) * g + bt


def _dot(a, b):
    return lax.dot_general(a, b, (((1,), (0,)), ((), ())),
                           preferred_element_type=jnp.float32)


def _dot_t(a, b):
    return lax.dot_general(a, b, (((0,), (0,)), ((), ())),
                           preferred_element_type=jnp.float32)


def _stats_of(y):
    return jnp.concatenate(
        [jnp.sum(y, axis=1, keepdims=True),
         jnp.sum(y * y, axis=1, keepdims=True)], axis=1)


def _acc_stats(s_ref, y):
    st = _stats_of(y)

    @pl.when(pl.program_id(0) == 0)
    def _():
        s_ref[...] = st

    @pl.when(pl.program_id(0) != 0)
    def _():
        s_ref[...] = s_ref[...] + st


def _knn_body(qT_ref, pT_ref, idx_ref, *, npts):
    qT = qT_ref[...]
    pT = pT_ref[...]
    sq_q = jnp.sum(qT * qT, axis=0)[:, None]
    sq_p = jnp.sum(pT * pT, axis=0)[None, :]
    d2 = sq_q + sq_p - 2.0 * _dot_t(qT, pT)
    iota = lax.broadcasted_iota(jnp.int32, d2.shape, 1)
    inf = jnp.float32(jnp.inf)
    cols = []
    for _ in range(_K):
        pos = jnp.argmin(d2, axis=1).astype(jnp.int32)
        cols.append(pos)
        d2 = jnp.where(iota == pos[:, None], inf, d2)
    idx_ref[...] = jnp.stack(cols, axis=1)


def _knn(qT, pT, tq=128):
    nq = qT.shape[1]
    npts = pT.shape[1]
    return pl.pallas_call(
        functools.partial(_knn_body, npts=npts),
        grid=(nq // tq,),
        in_specs=[
            pl.BlockSpec((3, tq), lambda i: (0, i)),
            pl.BlockSpec((3, npts), lambda i: (0, 0)),
        ],
        out_specs=pl.BlockSpec((tq, _K), lambda i: (i, 0)),
        out_shape=jax.ShapeDtypeStruct((nq, _K), jnp.int32),
        compiler_params=pltpu.CompilerParams(
            dimension_semantics=("parallel",)),
    )(qT, pT)


def _l1_body(pg_ref, q_ref, W_ref, y_ref, s_ref):
    x = pg_ref[...] - q_ref[...]
    y = _dot(W_ref[...], x)
    y_ref[...] = y
    _acc_stats(s_ref, y)


def _lmid_body(x_ref, s_ref, g_ref, bt_ref, W_ref, y_ref, so_ref, *, n):
    xn = jax.nn.relu(_apply_stats(x_ref[...], s_ref[...],
                                  g_ref[...], bt_ref[...], n))
    y = _dot(W_ref[...], xn)
    y_ref[...] = y
    _acc_stats(so_ref, y)


def _lmax_body(y_ref, add_ref, s_ref, g_ref, bt_ref, o_ref, *, n):
    z = _apply_stats(y_ref[...], s_ref[...], g_ref[...], bt_ref[...], n)
    z = z + add_ref[...]

    @pl.when(pl.program_id(0) == 0)
    def _():
        o_ref[...] = z

    @pl.when(pl.program_id(0) != 0)
    def _():
        o_ref[...] = jnp.maximum(o_ref[...], z)


def _seq_params(grid, cout):
    return dict(
        out_shape=(jax.ShapeDtypeStruct((cout, grid * _CB), jnp.float32),
                   jax.ShapeDtypeStruct((cout, 2), jnp.float32)),
        compiler_params=pltpu.CompilerParams(
            dimension_semantics=("arbitrary",)),
    )


def _mlp_max(pgT, qT, addT, blocks):
    grid = pgT.shape[1] // _CB
    n = float(pgT.shape[1])
    b0, b1, b2 = blocks
    c0, c1, c2 = b0["W"].shape[0], b1["W"].shape[0], b2["W"].shape[0]
    cs = lambda c: pl.BlockSpec((c, _CB), lambda i: (0, i))
    full = lambda r, c: pl.BlockSpec((r, c), lambda i: (0, 0))

    y1, s1 = pl.pallas_call(
        _l1_body, grid=(grid,),
        in_specs=[cs(3), full(3, _CB), full(c0, 3)],
        out_specs=(cs(c0), full(c0, 2)),
        **_seq_params(grid, c0))(pgT, qT, b0["W"])

    y2, s2 = pl.pallas_call(
        functools.partial(_lmid_body, n=n), grid=(grid,),
        in_specs=[cs(c0), full(c0, 2), full(c0, 1), full(c0, 1),
                  full(c1, c0)],
        out_specs=(cs(c1), full(c1, 2)),
        **_seq_params(grid, c1))(
            y1, s1, b0["g"][:, None], b0["bt"][:, None], b1["W"])

    y3, s3 = pl.pallas_call(
        functools.partial(_lmid_body, n=n), grid=(grid,),
        in_specs=[cs(c1), full(c1, 2), full(c1, 1), full(c1, 1),
                  full(c2, c1)],
        out_specs=(cs(c2), full(c2, 2)),
        **_seq_params(grid, c2))(
            y2, s2, b1["g"][:, None], b1["bt"][:, None], b2["W"])

    return pl.pallas_call(
        functools.partial(_lmax_body, n=n), grid=(grid,),
        in_specs=[cs(c2), cs(c2), full(c2, 2), full(c2, 1), full(c2, 1)],
        out_specs=pl.BlockSpec((c2, _CB), lambda i: (0, 0)),
        out_shape=jax.ShapeDtypeStruct((c2, _CB), jnp.float32),
        compiler_params=pltpu.CompilerParams(
            dimension_semantics=("arbitrary",)),
    )(y3, addT, s3, b2["g"][:, None], b2["bt"][:, None])


def _f1_body(fT_ref, W_ref, g_ref, bt_ref, out_ref):
    y = _dot(W_ref[...], fT_ref[...])
    out_ref[...] = jax.nn.relu(_bn_lanes(y, g_ref[...], bt_ref[...]))


def _f1(fT, W, g, bt):
    n = fT.shape[1]
    return pl.pallas_call(
        _f1_body,
        out_shape=jax.ShapeDtypeStruct((128, n), jnp.float32),
    )(fT, W, g, bt)


def _tail1_body(g_ref, gbn_ref, btbn_ref, Wp_ref, gp_ref, btp_ref,
                newf_ref, hpre_ref):
    newf = _bn_lanes(g_ref[...], gbn_ref[...], btbn_ref[...])
    newf_ref[...] = newf
    hpre_ref[...] = jax.nn.relu(
        _bn_lanes(_dot(Wp_ref[...], newf), gp_ref[...], btp_ref[...]))


def _tail2_body(m_ref, id_ref, gbn_ref, btbn_ref,
                Wf0_ref, gf0_ref, btf0_ref, Wf1_ref, gf1_ref, btf1_ref,
                out_ref):
    h = _bn_lanes(m_ref[...], gbn_ref[...], btbn_ref[...])
    h = jax.nn.relu(_bn_lanes(_dot(Wf0_ref[...], h),
                              gf0_ref[...], btf0_ref[...]))
    h = _bn_lanes(_dot(Wf1_ref[...], h), gf1_ref[...], btf1_ref[...])
    out_ref[...] = jax.nn.relu(id_ref[...] + h)


def kernel(p, f, index, offset, params):
    pm = params
    pT = p.T
    new_pT = jnp.take(pT, index, axis=1)
    m = new_pT.shape[1]

    f1T = _f1(f.T, pm["ds_conv1"]["W"],
              pm["ds_conv1"]["g"][:, None], pm["ds_conv1"]["bt"][:, None])

    idx1 = _knn(new_pT, pT)
    idx1_jm = idx1.T.reshape(-1)
    pgT = jnp.take(pT, idx1_jm, axis=1)
    f1gT = jnp.take(f1T, idx1_jm, axis=1)

    g_acc = _mlp_max(pgT, new_pT, f1gT,
                     [pm["ds_conv2_0"], pm["ds_conv2_1"], pm["ds_conv2_2"]])

    newfT, hpreT = pl.pallas_call(
        _tail1_body,
        out_shape=(jax.ShapeDtypeStruct((128, m), jnp.float32),
                   jax.ShapeDtypeStruct((128, m), jnp.float32)),
    )(g_acc, pm["ds_bn"]["g"][:, None], pm["ds_bn"]["bt"][:, None],
      pm["dsa0_pre"]["W"], pm["dsa0_pre"]["g"][:, None],
      pm["dsa0_pre"]["bt"][:, None])

    idx2 = _knn(new_pT, new_pT)
    idx2_jm = idx2.T.reshape(-1)
    pg2T = jnp.take(new_pT, idx2_jm, axis=1)
    hgT = jnp.take(hpreT, idx2_jm, axis=1)

    m_acc = _mlp_max(pg2T, new_pT, hgT,
                     [pm["pe_0"], pm["pe_1"], pm["pe_2"]])

    outT = pl.pallas_call(
        _tail2_body,
        out_shape=jax.ShapeDtypeStruct((128, m), jnp.float32),
    )(m_acc, newfT,
      pm["dsa0_bn"]["g"][:, None], pm["dsa0_bn"]["bt"][:, None],
      pm["dsa0_ffn0"]["W"], pm["dsa0_ffn0"]["g"][:, None],
      pm["dsa0_ffn0"]["bt"][:, None],
      pm["dsa0_ffn1"]["W"], pm["dsa0_ffn1"]["g"][:, None],
      pm["dsa0_ffn1"]["bt"][:, None])

    return new_pT.T, outT.T


# speedup vs baseline: 4.4442x; 1.2035x over previous
import functools
import jax
import jax.numpy as jnp
from jax import lax
from jax.experimental import pallas as pl
from jax.experimental.pallas import tpu as pltpu
from jax.experimental.pallas import tpu_sc as plsc


def _sc_gather(table, idx):
    v, d = table.shape
    b = idx.shape[0]
    info = plsc.get_sparse_core_info()
    nw = info.num_cores * info.num_subcores
    b_per_w = b // nw
    cb = min(b_per_w, max(8, (32768 // d) // 8 * 8))
    nchunks = b_per_w // cb
    mesh = plsc.VectorSubcoreMesh(core_axis_name="c", subcore_axis_name="s")

    @functools.partial(
        pl.kernel, mesh=mesh,
        out_type=jax.ShapeDtypeStruct((b, d), jnp.float32),
        scratch_types=[
            pltpu.VMEM((cb,), jnp.int32),
            pltpu.VMEM((cb, d), jnp.float32),
            pltpu.SemaphoreType.DMA,
        ],
    )
    def k(table_hbm, idx_hbm, out_hbm, idx_v, rows_v, sem):
        wid = lax.axis_index("s") * info.num_cores + lax.axis_index("c")
        base = wid * b_per_w
        for c in range(nchunks):
            off = base + c * cb
            pltpu.sync_copy(idx_hbm.at[pl.ds(off, cb)], idx_v)
            pltpu.async_copy(table_hbm.at[idx_v], rows_v, sem).wait()
            pltpu.sync_copy(rows_v, out_hbm.at[pl.ds(off, cb)])

    return k(table, idx)

_EPS = 1e-5
_K = 16
_CB = 4096


def _bn_lanes(x, g, bt):
    mean = jnp.mean(x, axis=1, keepdims=True)
    xc = x - mean
    var = jnp.mean(xc * xc, axis=1, keepdims=True)
    return xc / jnp.sqrt(var + _EPS) * g + bt


def _apply_stats(x, s, g, bt, n):
    mean = s[:, 0:1] / n
    var = s[:, 1:2] / n - mean * mean
    return (x - mean) / jnp.sqrt(var + _EPS) * g + bt


def _dot(a, b):
    return lax.dot_general(a, b, (((1,), (0,)), ((), ())),
                           preferred_element_type=jnp.float32)


def _dot_t(a, b):
    return lax.dot_general(a, b, (((0,), (0,)), ((), ())),
                           preferred_element_type=jnp.float32)


def _stats_of(y):
    return jnp.concatenate(
        [jnp.sum(y, axis=1, keepdims=True),
         jnp.sum(y * y, axis=1, keepdims=True)], axis=1)


def _acc_stats(s_ref, y):
    st = _stats_of(y)

    @pl.when(pl.program_id(0) == 0)
    def _():
        s_ref[...] = st

    @pl.when(pl.program_id(0) != 0)
    def _():
        s_ref[...] = s_ref[...] + st


def _knn_body(qT_ref, pT_ref, idx_ref, *, npts):
    qT = qT_ref[...]
    pT = pT_ref[...]
    sq_q = jnp.sum(qT * qT, axis=0)[:, None]
    sq_p = jnp.sum(pT * pT, axis=0)[None, :]
    d2 = sq_q + sq_p - 2.0 * _dot_t(qT, pT)
    iota = lax.broadcasted_iota(jnp.int32, d2.shape, 1)
    inf = jnp.float32(jnp.inf)
    cols = []
    for _ in range(_K):
        pos = jnp.argmin(d2, axis=1).astype(jnp.int32)
        cols.append(pos)
        d2 = jnp.where(iota == pos[:, None], inf, d2)
    idx_ref[...] = jnp.stack(cols, axis=1)


def _knn(qT, pT, tq=128):
    nq = qT.shape[1]
    npts = pT.shape[1]
    return pl.pallas_call(
        functools.partial(_knn_body, npts=npts),
        grid=(nq // tq,),
        in_specs=[
            pl.BlockSpec((3, tq), lambda i: (0, i)),
            pl.BlockSpec((3, npts), lambda i: (0, 0)),
        ],
        out_specs=pl.BlockSpec((tq, _K), lambda i: (i, 0)),
        out_shape=jax.ShapeDtypeStruct((nq, _K), jnp.int32),
        compiler_params=pltpu.CompilerParams(
            dimension_semantics=("parallel",)),
    )(qT, pT)


def _l1_body(pg_ref, q_ref, W_ref, y_ref, s_ref):
    x = pg_ref[...] - q_ref[...]
    y = _dot(W_ref[...], x)
    y_ref[...] = y
    _acc_stats(s_ref, y)


def _lmid_body(x_ref, s_ref, g_ref, bt_ref, W_ref, y_ref, so_ref, *, n):
    xn = jax.nn.relu(_apply_stats(x_ref[...], s_ref[...],
                                  g_ref[...], bt_ref[...], n))
    y = _dot(W_ref[...], xn)
    y_ref[...] = y
    _acc_stats(so_ref, y)


def _lmax_body(y_ref, add_ref, s_ref, g_ref, bt_ref, o_ref, *, n):
    z = _apply_stats(y_ref[...], s_ref[...], g_ref[...], bt_ref[...], n)
    z = z + add_ref[...]

    @pl.when(pl.program_id(0) == 0)
    def _():
        o_ref[...] = z

    @pl.when(pl.program_id(0) != 0)
    def _():
        o_ref[...] = jnp.maximum(o_ref[...], z)


def _seq_params(grid, cout):
    return dict(
        out_shape=(jax.ShapeDtypeStruct((cout, grid * _CB), jnp.float32),
                   jax.ShapeDtypeStruct((cout, 2), jnp.float32)),
        compiler_params=pltpu.CompilerParams(
            dimension_semantics=("arbitrary",)),
    )


def _mlp_max(pgT, qT, addT, blocks):
    grid = pgT.shape[1] // _CB
    n = float(pgT.shape[1])
    b0, b1, b2 = blocks
    c0, c1, c2 = b0["W"].shape[0], b1["W"].shape[0], b2["W"].shape[0]
    cs = lambda c: pl.BlockSpec((c, _CB), lambda i: (0, i))
    full = lambda r, c: pl.BlockSpec((r, c), lambda i: (0, 0))

    y1, s1 = pl.pallas_call(
        _l1_body, grid=(grid,),
        in_specs=[cs(3), full(3, _CB), full(c0, 3)],
        out_specs=(cs(c0), full(c0, 2)),
        **_seq_params(grid, c0))(pgT, qT, b0["W"])

    y2, s2 = pl.pallas_call(
        functools.partial(_lmid_body, n=n), grid=(grid,),
        in_specs=[cs(c0), full(c0, 2), full(c0, 1), full(c0, 1),
                  full(c1, c0)],
        out_specs=(cs(c1), full(c1, 2)),
        **_seq_params(grid, c1))(
            y1, s1, b0["g"][:, None], b0["bt"][:, None], b1["W"])

    y3, s3 = pl.pallas_call(
        functools.partial(_lmid_body, n=n), grid=(grid,),
        in_specs=[cs(c1), full(c1, 2), full(c1, 1), full(c1, 1),
                  full(c2, c1)],
        out_specs=(cs(c2), full(c2, 2)),
        **_seq_params(grid, c2))(
            y2, s2, b1["g"][:, None], b1["bt"][:, None], b2["W"])

    return pl.pallas_call(
        functools.partial(_lmax_body, n=n), grid=(grid,),
        in_specs=[cs(c2), cs(c2), full(c2, 2), full(c2, 1), full(c2, 1)],
        out_specs=pl.BlockSpec((c2, _CB), lambda i: (0, 0)),
        out_shape=jax.ShapeDtypeStruct((c2, _CB), jnp.float32),
        compiler_params=pltpu.CompilerParams(
            dimension_semantics=("arbitrary",)),
    )(y3, addT, s3, b2["g"][:, None], b2["bt"][:, None])


def _f1_body(fT_ref, W_ref, g_ref, bt_ref, out_ref):
    y = _dot(W_ref[...], fT_ref[...])
    out_ref[...] = jax.nn.relu(_bn_lanes(y, g_ref[...], bt_ref[...]))


def _f1(fT, W, g, bt):
    n = fT.shape[1]
    return pl.pallas_call(
        _f1_body,
        out_shape=jax.ShapeDtypeStruct((128, n), jnp.float32),
    )(fT, W, g, bt)


def _tail1_body(g_ref, gbn_ref, btbn_ref, Wp_ref, gp_ref, btp_ref,
                newf_ref, hpre_ref):
    newf = _bn_lanes(g_ref[...], gbn_ref[...], btbn_ref[...])
    newf_ref[...] = newf
    hpre_ref[...] = jax.nn.relu(
        _bn_lanes(_dot(Wp_ref[...], newf), gp_ref[...], btp_ref[...]))


def _tail2_body(m_ref, id_ref, gbn_ref, btbn_ref,
                Wf0_ref, gf0_ref, btf0_ref, Wf1_ref, gf1_ref, btf1_ref,
                out_ref):
    h = _bn_lanes(m_ref[...], gbn_ref[...], btbn_ref[...])
    h = jax.nn.relu(_bn_lanes(_dot(Wf0_ref[...], h),
                              gf0_ref[...], btf0_ref[...]))
    h = _bn_lanes(_dot(Wf1_ref[...], h), gf1_ref[...], btf1_ref[...])
    out_ref[...] = jax.nn.relu(id_ref[...] + h)


def kernel(p, f, index, offset, params):
    pm = params
    pT = p.T
    new_pT = jnp.take(pT, index, axis=1)
    m = new_pT.shape[1]

    f1T = _f1(f.T, pm["ds_conv1"]["W"],
              pm["ds_conv1"]["g"][:, None], pm["ds_conv1"]["bt"][:, None])

    idx1 = _knn(new_pT, pT)
    idx1_jm = idx1.T.reshape(-1)
    pgT = jnp.take(pT, idx1_jm, axis=1)
    f1gT = _sc_gather(f1T.T, idx1_jm).T

    g_acc = _mlp_max(pgT, new_pT, f1gT,
                     [pm["ds_conv2_0"], pm["ds_conv2_1"], pm["ds_conv2_2"]])

    newfT, hpreT = pl.pallas_call(
        _tail1_body,
        out_shape=(jax.ShapeDtypeStruct((128, m), jnp.float32),
                   jax.ShapeDtypeStruct((128, m), jnp.float32)),
    )(g_acc, pm["ds_bn"]["g"][:, None], pm["ds_bn"]["bt"][:, None],
      pm["dsa0_pre"]["W"], pm["dsa0_pre"]["g"][:, None],
      pm["dsa0_pre"]["bt"][:, None])

    idx2 = _knn(new_pT, new_pT)
    idx2_jm = idx2.T.reshape(-1)
    pg2T = jnp.take(new_pT, idx2_jm, axis=1)
    hgT = _sc_gather(hpreT.T, idx2_jm).T

    m_acc = _mlp_max(pg2T, new_pT, hgT,
                     [pm["pe_0"], pm["pe_1"], pm["pe_2"]])

    outT = pl.pallas_call(
        _tail2_body,
        out_shape=jax.ShapeDtypeStruct((128, m), jnp.float32),
    )(m_acc, newfT,
      pm["dsa0_bn"]["g"][:, None], pm["dsa0_bn"]["bt"][:, None],
      pm["dsa0_ffn0"]["W"], pm["dsa0_ffn0"]["g"][:, None],
      pm["dsa0_ffn0"]["bt"][:, None],
      pm["dsa0_ffn1"]["W"], pm["dsa0_ffn1"]["g"][:, None],
      pm["dsa0_ffn1"]["bt"][:, None])

    return new_pT.T, outT.T
